# baseline (speedup 1.0000x reference)
import functools

import jax
import jax.numpy as jnp
from jax.experimental import pallas as pl
from jax.experimental.pallas import tpu as pltpu

NUM_VECTOR = 8
LAYER_IDX = 16
BLK = 1024


def _body(nblk, x_ref, w_ref, b_ref, pool_ref, o_ref):
    s = pl.program_id(1)
    o_ref[0] = x_ref[0]

    @pl.when(s == 0)
    def _():
        act = x_ref[0, pl.ds(BLK - 1, 1), :]
        scores = jax.lax.dot_general(
            act, w_ref[...], (((1,), (0,)), ((), ())),
            precision=jax.lax.Precision.HIGHEST)
        scores = scores + b_ref[...]
        idx = jnp.argmax(scores[0, :], axis=0)
        keep = idx == NUM_VECTOR
        onehot = (jax.lax.broadcasted_iota(jnp.int32, (1, NUM_VECTOR), 1)
                  == idx).astype(jnp.float32)
        repl = jax.lax.dot_general(
            onehot, pool_ref[:, LAYER_IDX % 8, :], (((1,), (0,)), ((), ())),
            precision=jax.lax.Precision.HIGHEST)
        o_ref[0, pl.ds(BLK - 1, 1), :] = jnp.where(keep, act, repl)


def kernel(x, vector_pool, gate_W, gate_b):
    B, S, H = x.shape
    nblk = S // BLK
    gate_b2 = gate_b.reshape(1, -1)
    grid = (B, nblk)
    return pl.pallas_call(
        functools.partial(_body, nblk),
        grid=grid,
        in_specs=[
            pl.BlockSpec((1, BLK, H), lambda b, s: (b, nblk - 1 - s, 0)),
            pl.BlockSpec((H, NUM_VECTOR + 1), lambda b, s: (0, 0)),
            pl.BlockSpec((1, NUM_VECTOR + 1), lambda b, s: (0, 0)),
            pl.BlockSpec((NUM_VECTOR, 8, H), lambda b, s: (0, LAYER_IDX // 8, 0)),
        ],
        out_specs=pl.BlockSpec((1, BLK, H), lambda b, s: (b, nblk - 1 - s, 0)),
        out_shape=jax.ShapeDtypeStruct((B, S, H), x.dtype),
        compiler_params=pltpu.CompilerParams(
            dimension_semantics=("parallel", "arbitrary"),
            vmem_limit_bytes=100 * 1024 * 1024),
    )(x, gate_W, gate_b2, vector_pool)


# speedup vs baseline: 1.0486x; 1.0486x over previous
import functools

import jax
import jax.numpy as jnp
from jax.experimental import pallas as pl
from jax.experimental.pallas import tpu as pltpu

NUM_VECTOR = 8
LAYER_IDX = 16
BLK = 1024


def _body(nblk, x_ref, aux_ref, o_ref):
    s = pl.program_id(1)
    o_ref[0] = x_ref[0]

    @pl.when(s == 0)
    def _():
        act = x_ref[0, pl.ds(BLK - 1, 1), :]
        wt = aux_ref[NUM_VECTOR:2 * NUM_VECTOR + 1, :]
        scores = jax.lax.dot_general(
            act, wt, (((1,), (1,)), ((), ())),
            precision=jax.lax.Precision.HIGHEST)
        scores = scores + aux_ref[pl.ds(2 * NUM_VECTOR + 1, 1),
                                  pl.ds(0, NUM_VECTOR + 1)]
        idx = jnp.argmax(scores[0, :], axis=0)
        keep = idx == NUM_VECTOR
        onehot = (jax.lax.broadcasted_iota(jnp.int32, (1, NUM_VECTOR), 1)
                  == idx).astype(jnp.float32)
        repl = jax.lax.dot_general(
            onehot, aux_ref[:NUM_VECTOR, :], (((1,), (0,)), ((), ())),
            precision=jax.lax.Precision.HIGHEST)
        o_ref[0, pl.ds(BLK - 1, 1), :] = jnp.where(keep, act, repl)


def kernel(x, vector_pool, gate_W, gate_b):
    B, S, H = x.shape
    nblk = S // BLK
    naux = 2 * NUM_VECTOR + 2
    aux = jnp.concatenate([
        vector_pool[:, LAYER_IDX, :],
        gate_W.T,
        jnp.pad(gate_b, (0, H - gate_b.shape[0]))[None, :],
    ], axis=0)
    grid = (B, nblk)
    return pl.pallas_call(
        functools.partial(_body, nblk),
        grid=grid,
        in_specs=[
            pl.BlockSpec((1, BLK, H), lambda b, s: (b, nblk - 1 - s, 0)),
            pl.BlockSpec((naux, H), lambda b, s: (0, 0)),
        ],
        out_specs=pl.BlockSpec((1, BLK, H), lambda b, s: (b, nblk - 1 - s, 0)),
        out_shape=jax.ShapeDtypeStruct((B, S, H), x.dtype),
        compiler_params=pltpu.CompilerParams(
            dimension_semantics=("parallel", "arbitrary"),
            vmem_limit_bytes=100 * 1024 * 1024),
    )(x, aux)


# speedup vs baseline: 1.0499x; 1.0012x over previous
import functools

import jax
import jax.numpy as jnp
from jax.experimental import pallas as pl
from jax.experimental.pallas import tpu as pltpu

NUM_VECTOR = 8
LAYER_IDX = 16
BLK = 1024


def _body(nblk, x_ref, aux_ref, o_ref):
    s = pl.program_id(1)
    o_ref[0] = x_ref[0]

    @pl.when(s == 0)
    def _():
        act = x_ref[0, pl.ds(BLK - 1, 1), :]
        wt = aux_ref[NUM_VECTOR:2 * NUM_VECTOR + 1, :]
        scores = jax.lax.dot_general(
            act, wt, (((1,), (1,)), ((), ())),
            precision=jax.lax.Precision.HIGHEST)
        scores = scores + aux_ref[pl.ds(2 * NUM_VECTOR + 1, 1),
                                  pl.ds(0, NUM_VECTOR + 1)]
        idx = jnp.argmax(scores[0, :], axis=0)
        keep = idx == NUM_VECTOR
        onehot = (jax.lax.broadcasted_iota(jnp.int32, (1, NUM_VECTOR), 1)
                  == idx).astype(jnp.float32)
        repl = jax.lax.dot_general(
            onehot, aux_ref[:NUM_VECTOR, :], (((1,), (0,)), ((), ())),
            precision=jax.lax.Precision.HIGHEST)
        o_ref[0, pl.ds(BLK - 1, 1), :] = jnp.where(keep, act, repl)


def kernel(x, vector_pool, gate_W, gate_b):
    B, S, H = x.shape
    nblk = S // BLK
    naux = 2 * NUM_VECTOR + 2
    aux = jnp.concatenate([
        vector_pool[:, LAYER_IDX, :],
        gate_W.T,
        jnp.pad(gate_b, (0, H - gate_b.shape[0]))[None, :],
    ], axis=0)
    grid = (B, nblk)
    return pl.pallas_call(
        functools.partial(_body, nblk),
        grid=grid,
        in_specs=[
            pl.BlockSpec((1, BLK, H), lambda b, s: (b, nblk - 1 - s, 0)),
            pl.BlockSpec((naux, H), lambda b, s: (0, 0)),
        ],
        out_specs=pl.BlockSpec((1, BLK, H), lambda b, s: (b, nblk - 1 - s, 0)),
        out_shape=jax.ShapeDtypeStruct((B, S, H), x.dtype),
        compiler_params=pltpu.CompilerParams(
            dimension_semantics=("arbitrary", "arbitrary"),
            vmem_limit_bytes=100 * 1024 * 1024),
    )(x, aux)
